# baseline (speedup 1.0000x reference)
import functools

import jax
import jax.numpy as jnp
from jax import lax
from jax.experimental import pallas as pl
from jax.experimental.pallas import tpu as pltpu
from jax.experimental.pallas import tpu_sc as plsc

NFEAT = 32
B_TOTAL = 1048576
NW = 32
PPW = B_TOTAL // NW
C = 128
NG = C // 16
NCHUNK = PPW // C

LOC_BASE = 348160
LOC_ROWS = 1365
LOC_MIN_HI = 5
WPAD = C + 16
RIW = 32
NRI = C * 8 // RIW

_MAGIC3 = 2863311531


def _floor_f32(x):
    t = x.astype(jnp.int32)
    tf = t.astype(jnp.float32)
    t = jnp.where(tf > x, t - 1, t)
    return t, t.astype(jnp.float32)


def _level_offset(lvl):
    sh = (20 - 2 * lvl).astype(jnp.uint32)
    diff = jnp.uint32(1 << 20) - (jnp.uint32(1) << sh)
    return (diff * jnp.uint32(_MAGIC3)).astype(jnp.int32)


def _indices_weights(cu, cv, du):
    fp = jnp.minimum(jnp.maximum(du * 4096.0, 8.0), 4096.0)
    e = (lax.bitcast_convert_type(fp, jnp.int32) >> 23) - 127
    hi = jnp.minimum(e - 2, 9)
    lo = hi - 1
    rcp = lax.bitcast_convert_type((124 - lo) << 23, jnp.float32)
    wb = fp * rcp - 1.0
    idxs, wgts = [], []
    for lvl, blend in ((lo, 1.0 - wb), (hi, wb)):
        lr = jnp.int32(512) >> lvl
        log2lr = 9 - lvl
        off = _level_offset(lvl)
        lrf = lr.astype(jnp.float32)
        pfx = cu * lrf - 0.5
        pfy = cv * lrf - 0.5
        ix, fx = _floor_f32(pfx)
        iy, fy = _floor_f32(pfy)
        wx = pfx - fx
        wy = pfy - fy
        m = lr - 1
        px0 = ix & m
        px1 = (ix + 1) & m
        py0 = iy & m
        py1 = (iy + 1) & m
        rx0 = (px0 << log2lr) + off
        rx1 = (px1 << log2lr) + off
        wx0 = 1.0 - wx
        wy0 = 1.0 - wy
        idxs += [rx0 + py0, rx1 + py0, rx0 + py1, rx1 + py1]
        wgts += [wx0 * wy0 * blend, wx * wy0 * blend,
                 wx0 * wy * blend, wx * wy * blend]
    return idxs, wgts, hi


def _wtree(ws, gs):
    t0 = ws[0] * gs[0] + ws[1] * gs[1]
    t1 = ws[2] * gs[2] + ws[3] * gs[3]
    t2 = ws[4] * gs[4] + ws[5] * gs[5]
    t3 = ws[6] * gs[6] + ws[7] * gs[7]
    return (t0 + t1) + (t2 + t3)


def _sc_body(xt, enc, enc_loc, out,
             in_v, tbl_v, rem_idx, rem_wgt, rem_pid, out_v, rows_v,
             isem, gsem, osem):
    cid = lax.axis_index("c")
    sid = lax.axis_index("s")
    wid = sid * 2 + cid
    wbase = wid * PPW
    iota = lax.iota(jnp.int32, 16)

    pltpu.sync_copy(enc_loc, tbl_v)
    zero16 = jnp.zeros((16,), jnp.int32)
    for p in range(2):
        for r in range(NRI):
            for s in range(RIW // 16):
                rem_idx[p, r, pl.ds(s * 16, 16)] = zero16
    pltpu.async_copy(xt.at[pl.ds(0, 3), pl.ds(wbase, C)], in_v.at[0], isem)

    def process_remote(pi, po, n_rem, obase):
        nbd = (n_rem + 3) >> 2
        nba = (n_rem + 15) >> 4

        def rem_drain(k, carry2):
            pltpu.make_async_copy(
                enc.at[rem_idx.at[pi, k]],
                rows_v.at[pi, pl.ds(k * RIW, RIW)], gsem).wait()
            return carry2

        lax.fori_loop(0, nbd, rem_drain, 0)

        def rem_acc(rb, carry2):
            rbase = rb * 16
            valid = (rbase + iota) < n_rem
            pid = rem_pid[pi, pl.ds(rbase, 16)]
            ws = [rem_wgt[pi, j, pl.ds(rbase, 16)] for j in range(8)]
            rj = [(rbase + iota) * 8 + j for j in range(8)]
            rv = rows_v.at[pi]
            for f2 in range(0, NFEAT, 2):
                fa = jnp.full((16,), f2, jnp.int32)
                fb = jnp.full((16,), f2 + 1, jnp.int32)
                ga = [plsc.load_gather(rv, [rj[j], fa]) for j in range(8)]
                gb = [plsc.load_gather(rv, [rj[j], fb]) for j in range(8)]
                plsc.store_scatter(out_v.at[po], [pid, fa], _wtree(ws, ga),
                                   mask=valid)
                plsc.store_scatter(out_v.at[po], [pid, fb], _wtree(ws, gb),
                                   mask=valid)
            return carry2

        lax.fori_loop(0, nba, rem_acc, 0)
        pltpu.async_copy(out_v.at[po], out.at[pl.ds(obase, C)], osem)

    def chunk_body(ci, nrem_prev):
        par = ci & 1
        po = lax.rem(ci, 3)
        base = wbase + ci * C

        @pl.when(ci >= 3)
        def _():
            pltpu.make_async_copy(
                out_v.at[po], out.at[pl.ds(base, C)], osem).wait()

        pltpu.make_async_copy(
            xt.at[pl.ds(0, 3), pl.ds(base, C)], in_v.at[par], isem).wait()

        @pl.when(ci + 1 < NCHUNK)
        def _():
            pltpu.async_copy(
                xt.at[pl.ds(0, 3), pl.ds(base + C, C)], in_v.at[1 - par],
                isem)

        def group_body(g, n_rem):
            cu = in_v[par, 0, pl.ds(g * 16, 16)]
            cv = in_v[par, 1, pl.ds(g * 16, 16)]
            du = in_v[par, 2, pl.ds(g * 16, 16)]
            idxs, wgts, hi = _indices_weights(cu, cv, du)
            local = hi >= LOC_MIN_HI
            rem_i = 1 - local.astype(jnp.int32)
            remote = jnp.logical_not(local)
            nr_g = jnp.sum(rem_i)
            pvec = g * 16 + iota

            @pl.when(nr_g > 0)
            def _():
                pos = n_rem + plsc.cumsum(rem_i) - 1
                q = pos * 8
                for j in range(8):
                    qj = q + j
                    plsc.store_scatter(rem_idx.at[par],
                                       [qj >> 5, qj & (RIW - 1)],
                                       idxs[j], mask=remote)
                    plsc.store_scatter(rem_wgt.at[par],
                                       [jnp.full((16,), j, jnp.int32), pos],
                                       wgts[j], mask=remote)
                plsc.store_scatter(rem_pid.at[par], [pos], pvec, mask=remote)
            n_rem = n_rem + nr_g

            addr = [idxs[j] - LOC_BASE for j in range(8)]
            ov = out_v.at[po]
            for f2 in range(0, NFEAT, 2):
                foa = f2 * LOC_ROWS
                fob = foa + LOC_ROWS
                ga = [plsc.load_gather(tbl_v, [addr[j] + foa], mask=local)
                      for j in range(8)]
                gb = [plsc.load_gather(tbl_v, [addr[j] + fob], mask=local)
                      for j in range(8)]
                plsc.store_scatter(ov, [pvec, jnp.full((16,), f2, jnp.int32)],
                                   _wtree(wgts, ga))
                plsc.store_scatter(ov,
                                   [pvec, jnp.full((16,), f2 + 1, jnp.int32)],
                                   _wtree(wgts, gb))
            return n_rem

        n_rem = jnp.int32(0)
        for g in range(NG):
            n_rem = group_body(g, n_rem)

        @pl.when(ci >= 1)
        def _():
            process_remote(1 - par, lax.rem(ci + 2, 3), nrem_prev, base - C)

        nbd = (n_rem + 3) >> 2

        def rem_fire(k, carry2):
            pltpu.async_copy(enc.at[rem_idx.at[par, k]],
                             rows_v.at[par, pl.ds(k * RIW, RIW)], gsem)
            return carry2

        lax.fori_loop(0, nbd, rem_fire, 0)
        return n_rem

    nrem_last = lax.fori_loop(0, NCHUNK, chunk_body, jnp.int32(0))

    last = NCHUNK - 1
    process_remote(last & 1, lax.rem(jnp.int32(last), 3), nrem_last,
                   wbase + last * C)
    for _ in range(3):
        pltpu.make_async_copy(
            out_v.at[0], out.at[pl.ds(wbase, C)], osem).wait()


@functools.partial(
    pl.kernel,
    out_type=jax.ShapeDtypeStruct((B_TOTAL, NFEAT), jnp.float32),
    mesh=plsc.VectorSubcoreMesh(core_axis_name="c", subcore_axis_name="s"),
    compiler_params=pltpu.CompilerParams(
        needs_layout_passes=False, use_tc_tiling_on_sc=False
    ),
    scratch_types=[
        pltpu.VMEM((2, 3, C), jnp.float32),
        pltpu.VMEM((NFEAT * LOC_ROWS,), jnp.float32),
        pltpu.VMEM((2, NRI, RIW), jnp.int32),
        pltpu.VMEM((2, 8, WPAD), jnp.float32),
        pltpu.VMEM((2, WPAD), jnp.int32),
        pltpu.VMEM((3, C, NFEAT), jnp.float32),
        pltpu.VMEM((2, C * 8, NFEAT), jnp.float32),
        pltpu.SemaphoreType.DMA,
        pltpu.SemaphoreType.DMA,
        pltpu.SemaphoreType.DMA,
    ],
)
def _encode(xt, enc, enc_loc, out, *rest):
    _sc_body(xt, enc, enc_loc, out, *rest)


def kernel(x, encoder):
    xt = x.T
    enc_loc = encoder[LOC_BASE:].T.reshape(-1)
    return _encode(xt, encoder, enc_loc)


# speedup vs baseline: 1.4515x; 1.4515x over previous
import functools

import jax
import jax.numpy as jnp
from jax import lax
from jax.experimental import pallas as pl
from jax.experimental.pallas import tpu as pltpu
from jax.experimental.pallas import tpu_sc as plsc

NFEAT = 32
B_TOTAL = 1048576
NW = 32
PPW = B_TOTAL // NW
C = 128
NG = C // 16
NCHUNK = PPW // C

LOC_BASE = 348160
LOC_ROWS = 1365
LOC_MIN_HI = 5
WPAD = C + 16
RIW = 32
NRI = C * 8 // RIW

_MAGIC3 = 2863311531


def _floor_f32(x):
    t = x.astype(jnp.int32)
    tf = t.astype(jnp.float32)
    t = jnp.where(tf > x, t - 1, t)
    return t, t.astype(jnp.float32)


def _level_offset(lvl):
    sh = (20 - 2 * lvl).astype(jnp.uint32)
    diff = jnp.uint32(1 << 20) - (jnp.uint32(1) << sh)
    return (diff * jnp.uint32(_MAGIC3)).astype(jnp.int32)


def _indices_weights(cu, cv, du):
    fp = jnp.minimum(jnp.maximum(du * 4096.0, 8.0), 4096.0)
    e = (lax.bitcast_convert_type(fp, jnp.int32) >> 23) - 127
    hi = jnp.minimum(e - 2, 9)
    lo = hi - 1
    rcp = lax.bitcast_convert_type((124 - lo) << 23, jnp.float32)
    wb = fp * rcp - 1.0
    idxs, wgts = [], []
    for lvl, blend in ((lo, 1.0 - wb), (hi, wb)):
        lr = jnp.int32(512) >> lvl
        log2lr = 9 - lvl
        off = _level_offset(lvl)
        lrf = lr.astype(jnp.float32)
        pfx = cu * lrf - 0.5
        pfy = cv * lrf - 0.5
        ix, fx = _floor_f32(pfx)
        iy, fy = _floor_f32(pfy)
        wx = pfx - fx
        wy = pfy - fy
        m = lr - 1
        px0 = ix & m
        px1 = (ix + 1) & m
        py0 = iy & m
        py1 = (iy + 1) & m
        rx0 = (px0 << log2lr) + off
        rx1 = (px1 << log2lr) + off
        wx0 = 1.0 - wx
        wy0 = 1.0 - wy
        idxs += [rx0 + py0, rx1 + py0, rx0 + py1, rx1 + py1]
        wgts += [wx0 * wy0 * blend, wx * wy0 * blend,
                 wx0 * wy * blend, wx * wy * blend]
    return idxs, wgts, hi


def _wtree(ws, gs):
    t0 = ws[0] * gs[0] + ws[1] * gs[1]
    t1 = ws[2] * gs[2] + ws[3] * gs[3]
    t2 = ws[4] * gs[4] + ws[5] * gs[5]
    t3 = ws[6] * gs[6] + ws[7] * gs[7]
    return (t0 + t1) + (t2 + t3)


def _sc_body(xt, enc, enc_loc, out,
             in_v, tbl_v, rem_idx, rem_wgt, rem_pid, out_v, rows_v,
             isem, gsem, osem):
    cid = lax.axis_index("c")
    sid = lax.axis_index("s")
    wid = sid * 2 + cid
    wbase = wid * PPW
    iota = lax.iota(jnp.int32, 16)

    pltpu.sync_copy(enc_loc, tbl_v)
    zero16 = jnp.zeros((16,), jnp.int32)
    for p in range(2):
        for r in range(NRI):
            for s in range(RIW // 16):
                rem_idx[p, r, pl.ds(s * 16, 16)] = zero16
    pltpu.async_copy(xt.at[pl.ds(0, 3), pl.ds(wbase, C)], in_v.at[0], isem)

    def process_remote(pi, po, n_rem, obase):
        nbd = (n_rem + 3) >> 2
        nba = (n_rem + 15) >> 4

        def rem_drain(k, carry2):
            pltpu.make_async_copy(
                enc.at[rem_idx.at[pi, k]],
                rows_v.at[pi, pl.ds(k * RIW, RIW)], gsem).wait()
            return carry2

        lax.fori_loop(0, nbd, rem_drain, 0)

        def rem_acc(rb, carry2):
            rbase = rb * 16
            valid = (rbase + iota) < n_rem
            pid = rem_pid[pi, pl.ds(rbase, 16)]
            ws = [rem_wgt[pi, j, pl.ds(rbase, 16)] for j in range(8)]
            rj = [(rbase + iota) * 8 + j for j in range(8)]
            rv = rows_v.at[pi]
            for f2 in range(0, NFEAT, 2):
                fa = jnp.full((16,), f2, jnp.int32)
                fb = jnp.full((16,), f2 + 1, jnp.int32)
                ga = [plsc.load_gather(rv, [rj[j], fa]) for j in range(8)]
                gb = [plsc.load_gather(rv, [rj[j], fb]) for j in range(8)]
                plsc.store_scatter(out_v.at[po], [pid, fa], _wtree(ws, ga),
                                   mask=valid)
                plsc.store_scatter(out_v.at[po], [pid, fb], _wtree(ws, gb),
                                   mask=valid)
            return carry2

        lax.fori_loop(0, nba, rem_acc, 0)
        pltpu.async_copy(out_v.at[po], out.at[pl.ds(obase, C)], osem)

    def chunk_body(ci, nrem_prev):
        par = ci & 1
        po = lax.rem(ci, 3)
        base = wbase + ci * C

        @pl.when(ci >= 3)
        def _():
            pltpu.make_async_copy(
                out_v.at[po], out.at[pl.ds(base, C)], osem).wait()

        pltpu.make_async_copy(
            xt.at[pl.ds(0, 3), pl.ds(base, C)], in_v.at[par], isem).wait()

        @pl.when(ci + 1 < NCHUNK)
        def _():
            pltpu.async_copy(
                xt.at[pl.ds(0, 3), pl.ds(base + C, C)], in_v.at[1 - par],
                isem)

        def group_body(g, n_rem):
            cu = in_v[par, 0, pl.ds(g * 16, 16)]
            cv = in_v[par, 1, pl.ds(g * 16, 16)]
            du = in_v[par, 2, pl.ds(g * 16, 16)]
            idxs, wgts, hi = _indices_weights(cu, cv, du)
            local = hi >= LOC_MIN_HI
            rem_i = 1 - local.astype(jnp.int32)
            remote = jnp.logical_not(local)
            nr_g = jnp.sum(rem_i)
            pvec = g * 16 + iota

            @pl.when(nr_g > 0)
            def _():
                pos = n_rem + plsc.cumsum(rem_i) - 1
                q = pos * 8
                for j in range(8):
                    qj = q + j
                    plsc.store_scatter(rem_idx.at[par],
                                       [qj >> 5, qj & (RIW - 1)],
                                       idxs[j], mask=remote)
                    plsc.store_scatter(rem_wgt.at[par],
                                       [jnp.full((16,), j, jnp.int32), pos],
                                       wgts[j], mask=remote)
                plsc.store_scatter(rem_pid.at[par], [pos], pvec, mask=remote)
            n_rem = n_rem + nr_g

            addr = [idxs[j] - LOC_BASE for j in range(8)]
            ov = out_v.at[po]
            for f2 in range(0, NFEAT, 2):
                foa = f2 * LOC_ROWS
                fob = foa + LOC_ROWS
                ga = [plsc.load_gather(tbl_v, [addr[j] + foa], mask=local)
                      for j in range(8)]
                gb = [plsc.load_gather(tbl_v, [addr[j] + fob], mask=local)
                      for j in range(8)]
                plsc.store_scatter(ov, [pvec, jnp.full((16,), f2, jnp.int32)],
                                   _wtree(wgts, ga))
                plsc.store_scatter(ov,
                                   [pvec, jnp.full((16,), f2 + 1, jnp.int32)],
                                   _wtree(wgts, gb))
            return n_rem

        n_rem = lax.fori_loop(0, NG, group_body, jnp.int32(0))

        @pl.when(ci >= 1)
        def _():
            process_remote(1 - par, lax.rem(ci + 2, 3), nrem_prev, base - C)

        nbd = (n_rem + 3) >> 2

        def rem_fire(k, carry2):
            pltpu.async_copy(enc.at[rem_idx.at[par, k]],
                             rows_v.at[par, pl.ds(k * RIW, RIW)], gsem)
            return carry2

        lax.fori_loop(0, nbd, rem_fire, 0)
        return n_rem

    nrem_last = lax.fori_loop(0, NCHUNK, chunk_body, jnp.int32(0))

    last = NCHUNK - 1
    process_remote(last & 1, lax.rem(jnp.int32(last), 3), nrem_last,
                   wbase + last * C)
    for _ in range(3):
        pltpu.make_async_copy(
            out_v.at[0], out.at[pl.ds(wbase, C)], osem).wait()


@functools.partial(
    pl.kernel,
    out_type=jax.ShapeDtypeStruct((B_TOTAL, NFEAT), jnp.float32),
    mesh=plsc.VectorSubcoreMesh(core_axis_name="c", subcore_axis_name="s"),
    compiler_params=pltpu.CompilerParams(
        needs_layout_passes=False, use_tc_tiling_on_sc=False
    ),
    scratch_types=[
        pltpu.VMEM((2, 3, C), jnp.float32),
        pltpu.VMEM((NFEAT * LOC_ROWS,), jnp.float32),
        pltpu.VMEM((2, NRI, RIW), jnp.int32),
        pltpu.VMEM((2, 8, WPAD), jnp.float32),
        pltpu.VMEM((2, WPAD), jnp.int32),
        pltpu.VMEM((3, C, NFEAT), jnp.float32),
        pltpu.VMEM((2, C * 8, NFEAT), jnp.float32),
        pltpu.SemaphoreType.DMA,
        pltpu.SemaphoreType.DMA,
        pltpu.SemaphoreType.DMA,
    ],
)
def _encode(xt, enc, enc_loc, out, *rest):
    _sc_body(xt, enc, enc_loc, out, *rest)


def kernel(x, encoder):
    xt = x.T
    enc_loc = encoder[LOC_BASE:].T.reshape(-1)
    return _encode(xt, encoder, enc_loc)


# speedup vs baseline: 1.4707x; 1.0133x over previous
import functools

import jax
import jax.numpy as jnp
from jax import lax
from jax.experimental import pallas as pl
from jax.experimental.pallas import tpu as pltpu
from jax.experimental.pallas import tpu_sc as plsc

NFEAT = 32
B_TOTAL = 1048576
NW = 32
PPW = B_TOTAL // NW
C = 128
NG = C // 16
NCHUNK = PPW // C

LOC_BASE = 348160
LOC_ROWS = 1365
LOC_MIN_HI = 5
WPAD = C + 16
RIW = 32
NRI = C * 8 // RIW

_MAGIC3 = 2863311531


def _floor_f32(x):
    t = x.astype(jnp.int32)
    tf = t.astype(jnp.float32)
    t = jnp.where(tf > x, t - 1, t)
    return t, t.astype(jnp.float32)


def _level_offset(lvl):
    sh = (20 - 2 * lvl).astype(jnp.uint32)
    diff = jnp.uint32(1 << 20) - (jnp.uint32(1) << sh)
    return (diff * jnp.uint32(_MAGIC3)).astype(jnp.int32)


def _indices_weights(cu, cv, du):
    fp = jnp.minimum(jnp.maximum(du * 4096.0, 8.0), 4096.0)
    e = (lax.bitcast_convert_type(fp, jnp.int32) >> 23) - 127
    hi = jnp.minimum(e - 2, 9)
    lo = hi - 1
    rcp = lax.bitcast_convert_type((124 - lo) << 23, jnp.float32)
    wb = fp * rcp - 1.0
    idxs, wgts = [], []
    for lvl, blend in ((lo, 1.0 - wb), (hi, wb)):
        lr = jnp.int32(512) >> lvl
        log2lr = 9 - lvl
        off = _level_offset(lvl)
        lrf = lr.astype(jnp.float32)
        pfx = cu * lrf - 0.5
        pfy = cv * lrf - 0.5
        ix, fx = _floor_f32(pfx)
        iy, fy = _floor_f32(pfy)
        wx = pfx - fx
        wy = pfy - fy
        m = lr - 1
        px0 = ix & m
        px1 = (ix + 1) & m
        py0 = iy & m
        py1 = (iy + 1) & m
        rx0 = (px0 << log2lr) + off
        rx1 = (px1 << log2lr) + off
        wx0 = 1.0 - wx
        wy0 = 1.0 - wy
        idxs += [rx0 + py0, rx1 + py0, rx0 + py1, rx1 + py1]
        wgts += [wx0 * wy0 * blend, wx * wy0 * blend,
                 wx0 * wy * blend, wx * wy * blend]
    return idxs, wgts, hi


def _wtree(ws, gs):
    t0 = ws[0] * gs[0] + ws[1] * gs[1]
    t1 = ws[2] * gs[2] + ws[3] * gs[3]
    t2 = ws[4] * gs[4] + ws[5] * gs[5]
    t3 = ws[6] * gs[6] + ws[7] * gs[7]
    return (t0 + t1) + (t2 + t3)


def _sc_body(xt, enc, enc_loc, out,
             in_v, tbl_v, rem_idx, rem_wgt, rem_pid, out_v, rows_v,
             isem, gsem, osem):
    cid = lax.axis_index("c")
    sid = lax.axis_index("s")
    wid = sid * 2 + cid
    wbase = wid * PPW
    iota = lax.iota(jnp.int32, 16)

    pltpu.sync_copy(enc_loc, tbl_v)
    zero16 = jnp.zeros((16,), jnp.int32)
    for p in range(2):
        for r in range(NRI):
            for s in range(RIW // 16):
                rem_idx[p, r, pl.ds(s * 16, 16)] = zero16
    pltpu.async_copy(xt.at[pl.ds(0, 3), pl.ds(wbase, C)], in_v.at[0], isem)

    def process_remote(pi, po, n_rem, obase):
        nbd = (n_rem + 3) >> 2
        nba = (n_rem + 15) >> 4

        def rem_drain(k, carry2):
            pltpu.make_async_copy(
                enc.at[rem_idx.at[pi, k]],
                rows_v.at[pi, pl.ds(k * RIW, RIW)], gsem).wait()
            return carry2

        lax.fori_loop(0, nbd, rem_drain, 0)

        def rem_acc(rb, carry2):
            rbase = rb * 16
            valid = (rbase + iota) < n_rem
            pid = rem_pid[pi, pl.ds(rbase, 16)]
            ws = [rem_wgt[pi, j, pl.ds(rbase, 16)] for j in range(8)]
            rj = [(rbase + iota) * 8 + j for j in range(8)]
            rv = rows_v.at[pi]
            for f2 in range(0, NFEAT, 2):
                fa = jnp.full((16,), f2, jnp.int32)
                fb = jnp.full((16,), f2 + 1, jnp.int32)
                ga = [plsc.load_gather(rv, [rj[j], fa]) for j in range(8)]
                gb = [plsc.load_gather(rv, [rj[j], fb]) for j in range(8)]
                plsc.store_scatter(out_v.at[po], [pid, fa], _wtree(ws, ga),
                                   mask=valid)
                plsc.store_scatter(out_v.at[po], [pid, fb], _wtree(ws, gb),
                                   mask=valid)
            return carry2

        lax.fori_loop(0, nba, rem_acc, 0)
        pltpu.async_copy(out_v.at[po], out.at[pl.ds(obase, C)], osem)

    def chunk_body(ci, nrem_prev):
        par = ci & 1
        po = lax.rem(ci, 3)
        base = wbase + ci * C

        @pl.when(ci >= 3)
        def _():
            pltpu.make_async_copy(
                out_v.at[po], out.at[pl.ds(base, C)], osem).wait()

        pltpu.make_async_copy(
            xt.at[pl.ds(0, 3), pl.ds(base, C)], in_v.at[par], isem).wait()

        @pl.when(ci + 1 < NCHUNK)
        def _():
            pltpu.async_copy(
                xt.at[pl.ds(0, 3), pl.ds(base + C, C)], in_v.at[1 - par],
                isem)

        def group_body(g, n_rem):
            cu = in_v[par, 0, pl.ds(g * 16, 16)]
            cv = in_v[par, 1, pl.ds(g * 16, 16)]
            du = in_v[par, 2, pl.ds(g * 16, 16)]
            idxs, wgts, hi = _indices_weights(cu, cv, du)
            local = hi >= LOC_MIN_HI
            rem_i = 1 - local.astype(jnp.int32)
            remote = jnp.logical_not(local)

            pos = n_rem + plsc.cumsum(rem_i) - 1
            pvec = g * 16 + iota
            q = pos * 8
            for j in range(8):
                qj = q + j
                plsc.store_scatter(rem_idx.at[par],
                                   [qj >> 5, qj & (RIW - 1)],
                                   idxs[j], mask=remote)
                plsc.store_scatter(rem_wgt.at[par],
                                   [jnp.full((16,), j, jnp.int32), pos],
                                   wgts[j], mask=remote)
            plsc.store_scatter(rem_pid.at[par], [pos], pvec, mask=remote)
            n_rem = n_rem + jnp.sum(rem_i)

            addr = [idxs[j] - LOC_BASE for j in range(8)]
            ov = out_v.at[po]
            for f2 in range(0, NFEAT, 2):
                foff = (f2 // 2) * LOC_ROWS
                gw = [plsc.load_gather(tbl_v, [addr[j] + foff], mask=local)
                      for j in range(8)]
                ga = [lax.bitcast_convert_type(gw[j] << 16, jnp.float32)
                      for j in range(8)]
                gb = [lax.bitcast_convert_type(gw[j] & jnp.int32(-65536),
                                               jnp.float32)
                      for j in range(8)]
                plsc.store_scatter(ov, [pvec, jnp.full((16,), f2, jnp.int32)],
                                   _wtree(wgts, ga))
                plsc.store_scatter(ov,
                                   [pvec, jnp.full((16,), f2 + 1, jnp.int32)],
                                   _wtree(wgts, gb))
            return n_rem

        n_rem = lax.fori_loop(0, NG, group_body, jnp.int32(0))

        @pl.when(ci >= 1)
        def _():
            process_remote(1 - par, lax.rem(ci + 2, 3), nrem_prev, base - C)

        nbd = (n_rem + 3) >> 2

        def rem_fire(k, carry2):
            pltpu.async_copy(enc.at[rem_idx.at[par, k]],
                             rows_v.at[par, pl.ds(k * RIW, RIW)], gsem)
            return carry2

        lax.fori_loop(0, nbd, rem_fire, 0)
        return n_rem

    nrem_last = lax.fori_loop(0, NCHUNK, chunk_body, jnp.int32(0))

    last = NCHUNK - 1
    process_remote(last & 1, lax.rem(jnp.int32(last), 3), nrem_last,
                   wbase + last * C)
    for _ in range(3):
        pltpu.make_async_copy(
            out_v.at[0], out.at[pl.ds(wbase, C)], osem).wait()


@functools.partial(
    pl.kernel,
    out_type=jax.ShapeDtypeStruct((B_TOTAL, NFEAT), jnp.float32),
    mesh=plsc.VectorSubcoreMesh(core_axis_name="c", subcore_axis_name="s"),
    compiler_params=pltpu.CompilerParams(
        needs_layout_passes=False, use_tc_tiling_on_sc=False
    ),
    scratch_types=[
        pltpu.VMEM((2, 3, C), jnp.float32),
        pltpu.VMEM((NFEAT // 2 * LOC_ROWS,), jnp.int32),
        pltpu.VMEM((2, NRI, RIW), jnp.int32),
        pltpu.VMEM((2, 8, WPAD), jnp.float32),
        pltpu.VMEM((2, WPAD), jnp.int32),
        pltpu.VMEM((3, C, NFEAT), jnp.float32),
        pltpu.VMEM((2, C * 8, NFEAT), jnp.float32),
        pltpu.SemaphoreType.DMA,
        pltpu.SemaphoreType.DMA,
        pltpu.SemaphoreType.DMA,
    ],
)
def _encode(xt, enc, enc_loc, out, *rest):
    _sc_body(xt, enc, enc_loc, out, *rest)


def kernel(x, encoder):
    xt = x.T
    loc = encoder[LOC_BASE:].astype(jnp.bfloat16)
    u = lax.bitcast_convert_type(loc, jnp.uint16).astype(jnp.uint32)
    packed = u[:, 0::2] | (u[:, 1::2] << 16)
    enc_loc = lax.bitcast_convert_type(packed, jnp.int32).T.reshape(-1)
    return _encode(xt, encoder, enc_loc)
